# baseline (speedup 1.0000x reference)
import functools

import numpy as np
import jax
import jax.numpy as jnp
from jax import lax
from jax.experimental import pallas as pl
from jax.experimental.pallas import tpu as pltpu
from jax.experimental.pallas import tpu_sc as plsc

_SHAPE = (16384, 128)
_NUMEL = _SHAPE[0] * _SHAPE[1]
_COVERED = 64
_NBITS = 1


def _tf_rotl(x, d):
    return ((x << np.uint32(d)) | (x >> np.uint32(32 - d))).astype(np.uint32)


def _tf_raw(k1, k2, x1, x2):
    rot = [[13, 15, 26, 6], [17, 29, 16, 24]]
    ks = [np.uint32(k1), np.uint32(k2),
          np.uint32(np.uint32(k1) ^ np.uint32(k2) ^ np.uint32(0x1BD11BDA))]
    v0 = (x1 + ks[0]).astype(np.uint32)
    v1 = (x2 + ks[1]).astype(np.uint32)
    for i in range(5):
        for r in rot[i % 2]:
            v0 = (v0 + v1).astype(np.uint32)
            v1 = _tf_rotl(v1, r)
            v1 = (v1 ^ v0).astype(np.uint32)
        v0 = (v0 + ks[(i + 1) % 3]).astype(np.uint32)
        v1 = (v1 + ks[(i + 2) % 3] + np.uint32(i + 1)).astype(np.uint32)
    return v0, v1


def _tf_seed(s):
    return np.array([(s >> 32) & 0xffffffff, s & 0xffffffff], dtype=np.uint32)


def _tf_fold_in(key, d):
    sk = _tf_seed(d)
    o1, o2 = _tf_raw(key[0], key[1], sk[0:1], sk[1:2])
    return np.array([o1[0], o2[0]], dtype=np.uint32)


def _tf_split(key, n):
    b1, b2 = _tf_raw(key[0], key[1], np.zeros(n, np.uint32),
                     np.arange(n, dtype=np.uint32))
    return np.stack([b1, b2], axis=1)


def _tf_bits32(key, n):
    b1, b2 = _tf_raw(key[0], key[1], np.zeros(n, np.uint32),
                     np.arange(n, dtype=np.uint32))
    return (b1 ^ b2).astype(np.uint32)


def _tf_permutation(key, n):
    x = np.arange(n)
    num_rounds = int(np.ceil(3 * np.log(max(1, n)) /
                             np.log(np.iinfo(np.uint32).max)))
    for _ in range(num_rounds):
        ks = _tf_split(key, 2)
        key, subkey = ks[0], ks[1]
        x = x[np.argsort(_tf_bits32(subkey, n), kind="stable")]
    return x


def _flip_constants():
    k42 = _tf_seed(42)
    perm = _tf_permutation(_tf_fold_in(k42, 1), _NUMEL)
    idx = perm[:_COVERED].astype(np.int64)
    bit_keys = _tf_split(_tf_fold_in(k42, 2), _COVERED)
    bits = np.stack([_tf_permutation(bit_keys[i], 32)[:_NBITS]
                     for i in range(_COVERED)]).astype(np.uint32)
    mask = np.left_shift(np.uint32(1), bits).sum(axis=1, dtype=np.uint32)
    return idx, mask


_IDX, _MASK = _flip_constants()

_NC, _NS, _L = 2, 16, 16
_NW = _NC * _NS
_WROWS = _SHAPE[0] // _NW
_WELEMS = _WROWS * _SHAPE[1]
_NGROUPS = _COVERED // _L

_ORDER = np.argsort(_IDX, kind="stable")
_IDX1D = _IDX[_ORDER].astype(np.int32)
_MASK1D = _MASK[_ORDER].view(np.int32).copy()
_ROWS1D = (_IDX1D // _SHAPE[1]).astype(np.int32)
_COLS1D = (_IDX1D % _SHAPE[1]).astype(np.int32)

_mesh = plsc.VectorSubcoreMesh(core_axis_name="c", subcore_axis_name="s",
                               num_cores=_NC, num_subcores=_NS)


def _tc_copy_body(x_ref, o_ref):
    o_ref[...] = x_ref[...]


_TC_BLOCK = 1024

_tc_copy = pl.pallas_call(
    _tc_copy_body,
    grid=(_SHAPE[0] // _TC_BLOCK,),
    in_specs=[pl.BlockSpec((_TC_BLOCK, _SHAPE[1]), lambda i: (i, 0))],
    out_specs=pl.BlockSpec((_TC_BLOCK, _SHAPE[1]), lambda i: (i, 0)),
    out_shape=jax.ShapeDtypeStruct(_SHAPE, jnp.float32),
)


@functools.partial(
    pl.kernel,
    out_type=jax.ShapeDtypeStruct((_COVERED,), jnp.int32),
    mesh=_mesh,
    scratch_types=[
        pltpu.VMEM((_L,), jnp.int32),
        pltpu.VMEM((_L,), jnp.int32),
        pltpu.VMEM((_L,), jnp.float32),
        pltpu.VMEM((_L,), jnp.int32),
        pltpu.SemaphoreType.DMA,
    ],
)
def _sc_flips(x_hbm, idx_hbm, mask_hbm, out_hbm, idx16, m16, v16, o16, sem):
    wid = lax.axis_index("s") * _NC + lax.axis_index("c")

    @pl.when(wid < _NGROUPS)
    def _():
        base = wid * _L
        pltpu.sync_copy(idx_hbm.at[pl.ds(base, _L)], idx16)
        pltpu.sync_copy(mask_hbm.at[pl.ds(base, _L)], m16)
        pltpu.async_copy(x_hbm.at[idx16], v16, sem).wait()
        o16[...] = jax.lax.bitcast_convert_type(v16[...], jnp.int32) ^ m16[...]
        pltpu.sync_copy(o16, out_hbm.at[pl.ds(base, _L)])


_MB = 8


def _merge_body(rows_ref, cols_ref, fbits_ref, y_ref, o_ref):
    i = pl.program_id(0)
    rin = rows_ref[i] % _MB
    c = cols_ref[i]
    val = jax.lax.bitcast_convert_type(fbits_ref[i], jnp.float32)
    row_iota = jax.lax.broadcasted_iota(jnp.int32, (_MB, _SHAPE[1]), 0)
    col_iota = jax.lax.broadcasted_iota(jnp.int32, (_MB, _SHAPE[1]), 1)
    cond = (row_iota == rin) & (col_iota == c)
    o_ref[...] = jnp.where(cond, val, y_ref[...])


_tc_merge = pl.pallas_call(
    _merge_body,
    grid_spec=pltpu.PrefetchScalarGridSpec(
        num_scalar_prefetch=3,
        grid=(_COVERED,),
        in_specs=[pl.BlockSpec((_MB, _SHAPE[1]),
                               lambda i, rows, cols, fb: (rows[i] // _MB, 0))],
        out_specs=pl.BlockSpec((_MB, _SHAPE[1]),
                               lambda i, rows, cols, fb: (rows[i] // _MB, 0)),
    ),
    out_shape=jax.ShapeDtypeStruct(_SHAPE, jnp.float32),
    input_output_aliases={3: 0},
)


def kernel(x):
    y = _tc_copy(x)
    fbits = _sc_flips(x.reshape(-1), _IDX1D, _MASK1D)
    return _tc_merge(_ROWS1D, _COLS1D, fbits, y)


# speedup vs baseline: 1.8648x; 1.8648x over previous
import functools

import numpy as np
import jax
import jax.numpy as jnp
from jax import lax
from jax.experimental import pallas as pl
from jax.experimental.pallas import tpu as pltpu
from jax.experimental.pallas import tpu_sc as plsc

_SHAPE = (16384, 128)
_NUMEL = _SHAPE[0] * _SHAPE[1]
_COVERED = 64
_NBITS = 1


def _tf_rotl(x, d):
    return ((x << np.uint32(d)) | (x >> np.uint32(32 - d))).astype(np.uint32)


def _tf_raw(k1, k2, x1, x2):
    rot = [[13, 15, 26, 6], [17, 29, 16, 24]]
    ks = [np.uint32(k1), np.uint32(k2),
          np.uint32(np.uint32(k1) ^ np.uint32(k2) ^ np.uint32(0x1BD11BDA))]
    v0 = (x1 + ks[0]).astype(np.uint32)
    v1 = (x2 + ks[1]).astype(np.uint32)
    for i in range(5):
        for r in rot[i % 2]:
            v0 = (v0 + v1).astype(np.uint32)
            v1 = _tf_rotl(v1, r)
            v1 = (v1 ^ v0).astype(np.uint32)
        v0 = (v0 + ks[(i + 1) % 3]).astype(np.uint32)
        v1 = (v1 + ks[(i + 2) % 3] + np.uint32(i + 1)).astype(np.uint32)
    return v0, v1


def _tf_seed(s):
    return np.array([(s >> 32) & 0xffffffff, s & 0xffffffff], dtype=np.uint32)


def _tf_fold_in(key, d):
    sk = _tf_seed(d)
    o1, o2 = _tf_raw(key[0], key[1], sk[0:1], sk[1:2])
    return np.array([o1[0], o2[0]], dtype=np.uint32)


def _tf_split(key, n):
    b1, b2 = _tf_raw(key[0], key[1], np.zeros(n, np.uint32),
                     np.arange(n, dtype=np.uint32))
    return np.stack([b1, b2], axis=1)


def _tf_bits32(key, n):
    b1, b2 = _tf_raw(key[0], key[1], np.zeros(n, np.uint32),
                     np.arange(n, dtype=np.uint32))
    return (b1 ^ b2).astype(np.uint32)


def _tf_permutation(key, n):
    x = np.arange(n)
    num_rounds = int(np.ceil(3 * np.log(max(1, n)) /
                             np.log(np.iinfo(np.uint32).max)))
    for _ in range(num_rounds):
        ks = _tf_split(key, 2)
        key, subkey = ks[0], ks[1]
        x = x[np.argsort(_tf_bits32(subkey, n), kind="stable")]
    return x


def _flip_constants():
    k42 = _tf_seed(42)
    perm = _tf_permutation(_tf_fold_in(k42, 1), _NUMEL)
    idx = perm[:_COVERED].astype(np.int64)
    bit_keys = _tf_split(_tf_fold_in(k42, 2), _COVERED)
    bits = np.stack([_tf_permutation(bit_keys[i], 32)[:_NBITS]
                     for i in range(_COVERED)]).astype(np.uint32)
    mask = np.left_shift(np.uint32(1), bits).sum(axis=1, dtype=np.uint32)
    return idx, mask


_IDX, _MASK = _flip_constants()

_NC, _NS, _L = 2, 16, 16
_NW = _NC * _NS
_WROWS = _SHAPE[0] // _NW
_WELEMS = _WROWS * _SHAPE[1]
_NGROUPS = _COVERED // _L

_ORDER = np.argsort(_IDX, kind="stable")
_IDX1D = _IDX[_ORDER].astype(np.int32)
_MASK1D = _MASK[_ORDER].view(np.int32).copy()
_ROWS1D = (_IDX1D // _SHAPE[1]).astype(np.int32)
_COLS1D = (_IDX1D % _SHAPE[1]).astype(np.int32)

_mesh = plsc.VectorSubcoreMesh(core_axis_name="c", subcore_axis_name="s",
                               num_cores=_NC, num_subcores=_NS)


def _tc_copy_body(x_ref, o_ref):
    o_ref[...] = x_ref[...]


_TC_BLOCK = 1024

_tc_copy = pl.pallas_call(
    _tc_copy_body,
    grid=(_SHAPE[0] // _TC_BLOCK,),
    in_specs=[pl.BlockSpec((_TC_BLOCK, _SHAPE[1]), lambda i: (i, 0))],
    out_specs=pl.BlockSpec((_TC_BLOCK, _SHAPE[1]), lambda i: (i, 0)),
    out_shape=jax.ShapeDtypeStruct(_SHAPE, jnp.float32),
)


@functools.partial(
    pl.kernel,
    out_type=(),
    mesh=_mesh,
    scratch_types=[
        pltpu.VMEM((_L,), jnp.int32),
        pltpu.VMEM((_L,), jnp.int32),
        pltpu.VMEM((_L,), jnp.float32),
        pltpu.SemaphoreType.DMA,
    ],
)
def _sc_scatter(x_hbm, idx_hbm, mask_hbm, y_hbm, idx16, m16, v16, sem):
    wid = lax.axis_index("s") * _NC + lax.axis_index("c")

    @pl.when(wid < _NGROUPS)
    def _():
        base = wid * _L
        pltpu.sync_copy(idx_hbm.at[pl.ds(base, _L)], idx16)
        pltpu.sync_copy(mask_hbm.at[pl.ds(base, _L)], m16)
        pltpu.async_copy(x_hbm.at[idx16], v16, sem).wait()
        v16[...] = jax.lax.bitcast_convert_type(
            jax.lax.bitcast_convert_type(v16[...], jnp.int32) ^ m16[...],
            jnp.float32)
        pltpu.sync_copy(v16, y_hbm.at[idx16])


def kernel(x):
    y = _tc_copy(x)
    yf = jax.new_ref(y.reshape(-1))
    _sc_scatter(x.reshape(-1), _IDX1D, _MASK1D, yf)
    return yf[...].reshape(_SHAPE)


# speedup vs baseline: 2.3058x; 1.2365x over previous
import functools

import numpy as np
import jax
import jax.numpy as jnp
from jax import lax
from jax.experimental import pallas as pl
from jax.experimental.pallas import tpu as pltpu
from jax.experimental.pallas import tpu_sc as plsc

_SHAPE = (16384, 128)
_NUMEL = _SHAPE[0] * _SHAPE[1]
_COVERED = 64
_NBITS = 1


def _tf_rotl(x, d):
    return ((x << np.uint32(d)) | (x >> np.uint32(32 - d))).astype(np.uint32)


def _tf_raw(k1, k2, x1, x2):
    rot = [[13, 15, 26, 6], [17, 29, 16, 24]]
    ks = [np.uint32(k1), np.uint32(k2),
          np.uint32(np.uint32(k1) ^ np.uint32(k2) ^ np.uint32(0x1BD11BDA))]
    v0 = (x1 + ks[0]).astype(np.uint32)
    v1 = (x2 + ks[1]).astype(np.uint32)
    for i in range(5):
        for r in rot[i % 2]:
            v0 = (v0 + v1).astype(np.uint32)
            v1 = _tf_rotl(v1, r)
            v1 = (v1 ^ v0).astype(np.uint32)
        v0 = (v0 + ks[(i + 1) % 3]).astype(np.uint32)
        v1 = (v1 + ks[(i + 2) % 3] + np.uint32(i + 1)).astype(np.uint32)
    return v0, v1


def _tf_seed(s):
    return np.array([(s >> 32) & 0xffffffff, s & 0xffffffff], dtype=np.uint32)


def _tf_fold_in(key, d):
    sk = _tf_seed(d)
    o1, o2 = _tf_raw(key[0], key[1], sk[0:1], sk[1:2])
    return np.array([o1[0], o2[0]], dtype=np.uint32)


def _tf_split(key, n):
    b1, b2 = _tf_raw(key[0], key[1], np.zeros(n, np.uint32),
                     np.arange(n, dtype=np.uint32))
    return np.stack([b1, b2], axis=1)


def _tf_bits32(key, n):
    b1, b2 = _tf_raw(key[0], key[1], np.zeros(n, np.uint32),
                     np.arange(n, dtype=np.uint32))
    return (b1 ^ b2).astype(np.uint32)


def _tf_permutation(key, n):
    x = np.arange(n)
    num_rounds = int(np.ceil(3 * np.log(max(1, n)) /
                             np.log(np.iinfo(np.uint32).max)))
    for _ in range(num_rounds):
        ks = _tf_split(key, 2)
        key, subkey = ks[0], ks[1]
        x = x[np.argsort(_tf_bits32(subkey, n), kind="stable")]
    return x


def _flip_constants():
    k42 = _tf_seed(42)
    perm = _tf_permutation(_tf_fold_in(k42, 1), _NUMEL)
    idx = perm[:_COVERED].astype(np.int64)
    bit_keys = _tf_split(_tf_fold_in(k42, 2), _COVERED)
    bits = np.stack([_tf_permutation(bit_keys[i], 32)[:_NBITS]
                     for i in range(_COVERED)]).astype(np.uint32)
    mask = np.left_shift(np.uint32(1), bits).sum(axis=1, dtype=np.uint32)
    return idx, mask


_IDX, _MASK = _flip_constants()

_NC, _NS, _L = 2, 16, 16
_NW = _NC * _NS
_WROWS = _SHAPE[0] // _NW
_WELEMS = _WROWS * _SHAPE[1]
_NGROUPS = _COVERED // _L

_ORDER = np.argsort(_IDX, kind="stable")
_IDX1D = _IDX[_ORDER].astype(np.int32)
_MASK1D = _MASK[_ORDER].view(np.int32).copy()
_ROWS1D = (_IDX1D // _SHAPE[1]).astype(np.int32)
_COLS1D = (_IDX1D % _SHAPE[1]).astype(np.int32)

_mesh = plsc.VectorSubcoreMesh(core_axis_name="c", subcore_axis_name="s",
                               num_cores=_NC, num_subcores=_NS)


_tc_mesh = pltpu.create_tensorcore_mesh("tc")


@functools.partial(
    pl.kernel,
    out_type=(),
    mesh=_tc_mesh,
    scratch_types=[
        pltpu.VMEM((_NUMEL,), jnp.float32),
        pltpu.SemaphoreType.DMA,
        pltpu.SemaphoreType.DMA,
    ],
)
def _tc_copy_ref(x_hbm, y_hbm, buf, si, so):
    pltpu.async_copy(x_hbm, buf, si).wait()
    pltpu.async_copy(buf, y_hbm, so).wait()


@functools.partial(
    pl.kernel,
    out_type=(),
    mesh=_mesh,
    scratch_types=[
        pltpu.VMEM((_L,), jnp.int32),
        pltpu.VMEM((_L,), jnp.int32),
        pltpu.VMEM((_L,), jnp.float32),
        pltpu.SemaphoreType.DMA,
    ],
)
def _sc_scatter(x_hbm, idx_hbm, mask_hbm, y_hbm, idx16, m16, v16, sem):
    wid = lax.axis_index("s") * _NC + lax.axis_index("c")

    @pl.when(wid < _NGROUPS)
    def _():
        base = wid * _L
        pltpu.sync_copy(idx_hbm.at[pl.ds(base, _L)], idx16)
        pltpu.sync_copy(mask_hbm.at[pl.ds(base, _L)], m16)
        pltpu.async_copy(x_hbm.at[idx16], v16, sem).wait()
        v16[...] = jax.lax.bitcast_convert_type(
            jax.lax.bitcast_convert_type(v16[...], jnp.int32) ^ m16[...],
            jnp.float32)
        pltpu.sync_copy(v16, y_hbm.at[idx16])


def kernel(x):
    xf = x.reshape(-1)
    yf = jax.empty_ref(jax.ShapeDtypeStruct((_NUMEL,), jnp.float32))
    _tc_copy_ref(xf, yf)
    _sc_scatter(xf, _IDX1D, _MASK1D, yf)
    return yf[...].reshape(_SHAPE)
